# Initial kernel scaffold; baseline (speedup 1.0000x reference)
import jax
import jax.numpy as jnp
from jax.experimental import pallas as pl


def kernel(k, v, mem_index, key_buffer, value_buffer):
    raise NotImplementedError("write your pallas kernel here")


# speedup vs baseline: 2.1153x; 2.1153x over previous
import jax
import jax.numpy as jnp
from jax.experimental import pallas as pl
from jax.experimental.pallas import tpu as pltpu


def _body(idx_ref, k_ref, v_ref, kb_ref, vb_ref, out_ref, *, bm, nb):
    i = pl.program_id(0)
    out_ref[0] = kb_ref[...]
    out_ref[1] = vb_ref[...]
    base = i * bm
    for b in range(nb):
        idx = idx_ref[b]
        rel = idx - base

        @pl.when((idx >= base) & (idx < base + bm))
        def _():
            out_ref[0, rel] = k_ref[b]
            out_ref[1, rel] = v_ref[b]


def kernel(k, v, mem_index, key_buffer, value_buffer):
    m, h, d = key_buffer.shape
    nb = k.shape[0]
    kh = k.reshape(nb, h, d)
    vh = v.reshape(nb, h, d)
    bm = min(1024, m)
    grid = (m // bm,)

    import functools
    body = functools.partial(_body, bm=bm, nb=nb)
    return pl.pallas_call(
        body,
        grid=grid,
        in_specs=[
            pl.BlockSpec(memory_space=pltpu.SMEM),
            pl.BlockSpec((nb, h, d), lambda i: (0, 0, 0)),
            pl.BlockSpec((nb, h, d), lambda i: (0, 0, 0)),
            pl.BlockSpec((bm, h, d), lambda i: (i, 0, 0)),
            pl.BlockSpec((bm, h, d), lambda i: (i, 0, 0)),
        ],
        out_specs=pl.BlockSpec((2, bm, h, d), lambda i: (0, i, 0, 0)),
        out_shape=jax.ShapeDtypeStruct((2, m, h, d), key_buffer.dtype),
    )(mem_index.astype(jnp.int32), kh, vh, key_buffer, value_buffer)
